# Initial kernel scaffold; baseline (speedup 1.0000x reference)
import functools

import jax
import jax.numpy as jnp
from jax import lax
from jax.experimental import pallas as pl
from jax.experimental.pallas import tpu as pltpu
from jax.experimental.pallas import tpu_sc as plsc

_N = 10000
_E = 320000
_D = 128
_NC = 2
_NS = 16
_NW = _NC * _NS
_EPT = _E // _NW
_CH = 80
_NCH = _EPT // _CH
_ZR = 125
_RPT = _N // _NS
_BN = 1000
_GRID = _N // _BN


def _edge_body(with_denom, *refs):
    if with_denom:
        (y_hbm, src_hbm, dst_hbm, w_hbm, msg_hbm, den_hbm,
         src_v, dst_v, w_v, rbuf, zbuf, den_v, sem, msg_sp) = refs
    else:
        (y_hbm, src_hbm, dst_hbm, w_hbm, msg_hbm,
         src_v, dst_v, w_v, rbuf, zbuf, sem, msg_sp) = refs
    ci = lax.axis_index("c")
    si = lax.axis_index("s")
    wid = ci * _NS + si

    pltpu.sync_copy(src_hbm.at[wid], src_v)
    pltpu.sync_copy(dst_hbm.at[wid], dst_v)
    pltpu.sync_copy(w_hbm.at[wid], w_v)

    zero16 = jnp.zeros((16,), jnp.float32)

    def _zrow(i, _):
        for r in range(_D // 16):
            zbuf[i, pl.ds(r * 16, 16)] = zero16
        return ()

    lax.fori_loop(0, _ZR, _zrow, ())
    for b in range(_RPT // _ZR):
        pltpu.sync_copy(zbuf, msg_sp.at[pl.ds(si * _RPT + b * _ZR, _ZR)])
    if with_denom:
        def _zden(i, _):
            den_v[pl.ds(i * 16, 16)] = zero16
            return ()
        lax.fori_loop(0, _N // 16, _zden, ())
    plsc.subcore_barrier()

    def _chunk(c, _):
        pltpu.async_copy(y_hbm.at[src_v.at[pl.ds(c * _CH, _CH)]], rbuf, sem).wait()

        def _scale(e, _):
            wv = plsc.load_gather(w_v, [jnp.full((16,), c * _CH + e, jnp.int32)])
            for r in range(_D // 16):
                rbuf[e, pl.ds(r * 16, 16)] = rbuf[e, pl.ds(r * 16, 16)] * wv
            return ()

        lax.fori_loop(0, _CH, _scale, ())

        if with_denom:
            for j in range(_CH // 16):
                dstv = dst_v[c, pl.ds(j * 16, 16)]
                wv16 = w_v[pl.ds(c * _CH + j * 16, 16)]
                plsc.addupdate_scatter(den_v, [dstv], wv16)

        pltpu.sync_copy(rbuf, msg_sp.at[dst_v.at[c]], add=True)
        return ()

    lax.fori_loop(0, _NCH, _chunk, ())

    plsc.subcore_barrier()
    pltpu.sync_copy(msg_sp.at[pl.ds(si * _RPT, _RPT)],
                    msg_hbm.at[ci, pl.ds(si * _RPT, _RPT)])
    if with_denom:
        pltpu.sync_copy(den_v, den_hbm.at[wid])


def _make_edge(with_denom):
    mesh = plsc.VectorSubcoreMesh(core_axis_name="c", subcore_axis_name="s")
    out_type = [jax.ShapeDtypeStruct((_NC, _N, _D), jnp.float32)]
    if with_denom:
        out_type.append(jax.ShapeDtypeStruct((_NW, _N), jnp.float32))
    scratch = [
        pltpu.VMEM((_EPT,), jnp.int32),
        pltpu.VMEM((_NCH, _CH), jnp.int32),
        pltpu.VMEM((_EPT,), jnp.float32),
        pltpu.VMEM((_CH, _D), jnp.float32),
        pltpu.VMEM((_ZR, _D), jnp.float32),
    ]
    if with_denom:
        scratch.append(pltpu.VMEM((_N,), jnp.float32))
    scratch += [
        pltpu.SemaphoreType.DMA,
        pltpu.VMEM_SHARED((_N, _D), jnp.float32),
    ]
    return pl.kernel(functools.partial(_edge_body, with_denom),
                     out_type=out_type, mesh=mesh, scratch_types=scratch)


_edge_k_denom = _make_edge(True)
_edge_k = _make_edge(False)


def _dotT(x, w):
    return lax.dot_general(x, w, (((1,), (1,)), ((), ())),
                           preferred_element_type=jnp.float32,
                           precision=lax.Precision.HIGHEST)


def _tc_in_body(x_ref, w0_ref, w1_ref, b0_ref, b1_ref, z0_ref, y_ref):
    x = x_ref[...]
    z0_ref[...] = _dotT(x, w0_ref[...]) + b0_ref[...] + b1_ref[...]
    y_ref[...] = _dotT(x, w1_ref[...])


def _rinv_block(den_ref):
    i = pl.program_id(0)
    den = den_ref[:, pl.ds(i * _BN, _BN)]
    return 1.0 / (jnp.sum(den, axis=0)[:, None] + 1e-9)


def _tc_mid_body(z0_ref, agg_ref, den_ref, w0_ref, w1_ref, b0_ref, b1_ref,
                 z02_ref, y2_ref):
    rinv = _rinv_block(den_ref)
    h = jnp.maximum(z0_ref[...] + (agg_ref[0] + agg_ref[1]) * rinv, 0.0)
    z02_ref[...] = _dotT(h, w0_ref[...]) + b0_ref[...] + b1_ref[...]
    y2_ref[...] = _dotT(h, w1_ref[...])


def _tc_out_body(z0_ref, agg_ref, den_ref, out_ref):
    rinv = _rinv_block(den_ref)
    out_ref[...] = jnp.maximum(z0_ref[...] + (agg_ref[0] + agg_ref[1]) * rinv, 0.0)


_spec_rows = pl.BlockSpec((_BN, _D), lambda i: (i, 0))
_spec_w = pl.BlockSpec((_D, _D), lambda i: (0, 0))
_spec_b = pl.BlockSpec((1, _D), lambda i: (0, 0))
_spec_agg = pl.BlockSpec((_NC, _BN, _D), lambda i: (0, i, 0))
_spec_den = pl.BlockSpec((_NW, _N), lambda i: (0, 0))

_tc_in = pl.pallas_call(
    _tc_in_body,
    grid=(_GRID,),
    in_specs=[_spec_rows, _spec_w, _spec_w, _spec_b, _spec_b],
    out_specs=[_spec_rows, _spec_rows],
    out_shape=[jax.ShapeDtypeStruct((_N, _D), jnp.float32)] * 2,
)

_tc_mid = pl.pallas_call(
    _tc_mid_body,
    grid=(_GRID,),
    in_specs=[_spec_rows, _spec_agg, _spec_den, _spec_w, _spec_w, _spec_b, _spec_b],
    out_specs=[_spec_rows, _spec_rows],
    out_shape=[jax.ShapeDtypeStruct((_N, _D), jnp.float32)] * 2,
)

_tc_out = pl.pallas_call(
    _tc_out_body,
    grid=(_GRID,),
    in_specs=[_spec_rows, _spec_agg, _spec_den],
    out_specs=_spec_rows,
    out_shape=jax.ShapeDtypeStruct((_N, _D), jnp.float32),
)


def kernel(x, edge_index, edge_weight, W0, b0, W1, b1):
    dst = edge_index[0]
    src = edge_index[1]
    srcs = src.reshape(_NW, _EPT)
    dsts = dst.reshape(_NW, _NCH, _CH)
    ws = edge_weight.reshape(_NW, _EPT)
    b0r = b0.reshape(1, _D)
    b1r = b1.reshape(1, _D)

    z0, y = _tc_in(x, W0, W1, b0r, b1r)
    msg1, den = _edge_k_denom(y, srcs, dsts, ws)
    z02, y2 = _tc_mid(z0, msg1, den, W0, W1, b0r, b1r)
    msg2 = _edge_k(y2, srcs, dsts, ws)
    return _tc_out(z02, msg2, den)


# speedup vs baseline: 9.4069x; 9.4069x over previous
import functools

import jax
import jax.numpy as jnp
from jax import lax
from jax.experimental import pallas as pl
from jax.experimental.pallas import tpu as pltpu
from jax.experimental.pallas import tpu_sc as plsc

_N = 10000
_E = 320000
_D = 128
_NC = 2
_NS = 16
_NW = _NC * _NS
_EPT = _E // _NW
_CH = 80
_NCH = _EPT // _CH
_ZR = 80
_RB = 640
_BN = 1024
_GRID = (_N + _BN - 1) // _BN
_DENP = _BN * _GRID


_GDN = lax.GatherDimensionNumbers(offset_dims=(), collapsed_slice_dims=(0,),
                                  start_index_map=(0,))


def _lane_bcast(v, lane):
    idx = jnp.full((16, 1), lane, jnp.int32)
    return lax.gather(v, idx, _GDN, (1,),
                      mode=lax.GatherScatterMode.PROMISE_IN_BOUNDS)


def _edge_body(with_denom, *refs):
    if with_denom:
        (y_hbm, src_hbm, dst_hbm, w_hbm, msg_hbm, den_hbm,
         src_v, dst_v, w_v, rbuf, zv, sem, msg_sp, den_sp) = refs
    else:
        (y_hbm, src_hbm, dst_hbm, w_hbm, msg_hbm,
         src_v, dst_v, w_v, rbuf, sem, msg_sp) = refs
    ci = lax.axis_index("c")
    si = lax.axis_index("s")
    wid = ci * _NS + si

    pltpu.sync_copy(src_hbm.at[pl.ds(wid * _EPT, _EPT)], src_v)
    pltpu.sync_copy(dst_hbm.at[wid], dst_v)
    pltpu.sync_copy(w_hbm.at[pl.ds(wid * _EPT, _EPT)], w_v)

    zero16 = jnp.zeros((16,), jnp.float32)

    def _zrow(i, _):
        for r in range(_D // 16):
            rbuf[i, pl.ds(r * 16, 16)] = zero16
        return ()

    lax.fori_loop(0, _ZR, _zrow, ())

    @pl.when(si < _NS - 1)
    def _():
        for b in range(_RB // _ZR):
            pltpu.sync_copy(rbuf, msg_sp.at[pl.ds(si * _RB + b * _ZR, _ZR)])

    @pl.when(si == _NS - 1)
    def _():
        base = (_NS - 1) * _RB
        for b in range(400 // _ZR):
            pltpu.sync_copy(rbuf, msg_sp.at[pl.ds(base + b * _ZR, _ZR)])

    if with_denom:
        def _zv(i, _):
            zv[pl.ds(i * 16, 16)] = zero16
            return ()
        lax.fori_loop(0, _RB // 16, _zv, ())

        @pl.when(si < _NS - 1)
        def _():
            pltpu.sync_copy(zv, den_sp.at[pl.ds(si * _RB, _RB)])

        @pl.when(si == _NS - 1)
        def _():
            pltpu.sync_copy(zv.at[pl.ds(0, 400)],
                            den_sp.at[pl.ds((_NS - 1) * _RB, 400)])
    plsc.subcore_barrier()

    def _chunk(c, _):
        pltpu.async_copy(y_hbm.at[src_v.at[pl.ds(c * _CH, _CH)]], rbuf, sem).wait()

        def _scale(j, _):
            wv16 = w_v[pl.ds(c * _CH + j * 16, 16)]
            for e16 in range(16):
                e = j * 16 + e16
                wb = _lane_bcast(wv16, e16)
                for r in range(_D // 16):
                    rbuf[e, pl.ds(r * 16, 16)] = rbuf[e, pl.ds(r * 16, 16)] * wb
            return ()

        lax.fori_loop(0, _CH // 16, _scale, ())

        if with_denom:
            pltpu.sync_copy(w_v.at[pl.ds(c * _CH, _CH)],
                            den_sp.at[dst_v.at[c]], add=True)

        pltpu.sync_copy(rbuf, msg_sp.at[dst_v.at[c]], add=True)
        return ()

    lax.fori_loop(0, _NCH, _chunk, ())

    plsc.subcore_barrier()

    def _bounce_rows(lo):
        pltpu.sync_copy(msg_sp.at[pl.ds(lo, _ZR)], rbuf)
        pltpu.sync_copy(rbuf, msg_hbm.at[ci, pl.ds(lo, _ZR)])

    @pl.when(si < _NS - 1)
    def _():
        for b in range(_RB // _ZR):
            _bounce_rows(si * _RB + b * _ZR)

    @pl.when(si == _NS - 1)
    def _():
        base = (_NS - 1) * _RB
        for b in range(400 // _ZR):
            _bounce_rows(base + b * _ZR)

    if with_denom:
        @pl.when(si < _NS - 1)
        def _():
            pltpu.sync_copy(den_sp.at[pl.ds(si * _RB, _RB)], zv)
            pltpu.sync_copy(zv, den_hbm.at[pl.ds(ci * _DENP + si * _RB, _RB)])

        @pl.when(si == _NS - 1)
        def _():
            pltpu.sync_copy(den_sp.at[pl.ds((_NS - 1) * _RB, 400)],
                            zv.at[pl.ds(0, 400)])
            pltpu.sync_copy(zv.at[pl.ds(0, 400)],
                            den_hbm.at[pl.ds(ci * _DENP + (_NS - 1) * _RB, 400)])
            pltpu.sync_copy(zv.at[pl.ds(0, 240)],
                            den_hbm.at[pl.ds(ci * _DENP + _N, 240)])


def _make_edge(with_denom):
    mesh = plsc.VectorSubcoreMesh(core_axis_name="c", subcore_axis_name="s")
    out_type = [jax.ShapeDtypeStruct((_NC, _N, _D), jnp.float32)]
    if with_denom:
        out_type.append(jax.ShapeDtypeStruct((_NC * _DENP,), jnp.float32))
    scratch = [
        pltpu.VMEM((_EPT,), jnp.int32),
        pltpu.VMEM((_NCH, _CH), jnp.int32),
        pltpu.VMEM((_EPT,), jnp.float32),
        pltpu.VMEM((_CH, _D), jnp.float32),
    ]
    if with_denom:
        scratch.append(pltpu.VMEM((_RB,), jnp.float32))
    scratch += [
        pltpu.SemaphoreType.DMA,
        pltpu.VMEM_SHARED((_N, _D), jnp.float32),
    ]
    if with_denom:
        scratch.append(pltpu.VMEM_SHARED((_N,), jnp.float32))
    return pl.kernel(functools.partial(_edge_body, with_denom),
                     out_type=out_type, mesh=mesh, scratch_types=scratch)


_edge_k_denom = _make_edge(True)


def _dotT(x, w):
    return lax.dot_general(x, w, (((1,), (1,)), ((), ())),
                           preferred_element_type=jnp.float32,
                           precision=lax.Precision.HIGHEST)


def _tc_in_body(x_ref, w0_ref, w1_ref, b0_ref, b1_ref, z0_ref, y_ref):
    x = x_ref[...]
    z0_ref[...] = _dotT(x, w0_ref[...]) + b0_ref[...] + b1_ref[...]
    y_ref[...] = _dotT(x, w1_ref[...])


def _rinv_block(den_ref):
    i = pl.program_id(0)
    den = den_ref[:, pl.ds(i * _BN, _BN)]
    return 1.0 / (jnp.sum(den, axis=0)[:, None] + 1e-9)


def _tc_mid_body(z0_ref, agg_ref, den_ref, w0_ref, w1_ref, b0_ref, b1_ref,
                 z02_ref, y2_ref):
    rinv = _rinv_block(den_ref)
    h = jnp.maximum(z0_ref[...] + (agg_ref[0] + agg_ref[1]) * rinv, 0.0)
    z02_ref[...] = _dotT(h, w0_ref[...]) + b0_ref[...] + b1_ref[...]
    y2_ref[...] = _dotT(h, w1_ref[...])


def _tc_out_body(z0_ref, agg_ref, den_ref, out_ref):
    rinv = _rinv_block(den_ref)
    out_ref[...] = jnp.maximum(z0_ref[...] + (agg_ref[0] + agg_ref[1]) * rinv, 0.0)


_spec_rows = pl.BlockSpec((_BN, _D), lambda i: (i, 0))
_spec_w = pl.BlockSpec((_D, _D), lambda i: (0, 0))
_spec_b = pl.BlockSpec((1, _D), lambda i: (0, 0))
_spec_agg = pl.BlockSpec((_NC, _BN, _D), lambda i: (0, i, 0))
_spec_den = pl.BlockSpec((_NC, _DENP), lambda i: (0, 0))

_tc_in = pl.pallas_call(
    _tc_in_body,
    grid=(_GRID,),
    in_specs=[_spec_rows, _spec_w, _spec_w, _spec_b, _spec_b],
    out_specs=[_spec_rows, _spec_rows],
    out_shape=[jax.ShapeDtypeStruct((_N, _D), jnp.float32)] * 2,
)

_tc_mid = pl.pallas_call(
    _tc_mid_body,
    grid=(_GRID,),
    in_specs=[_spec_rows, _spec_agg, _spec_den, _spec_w, _spec_w, _spec_b, _spec_b],
    out_specs=[_spec_rows, _spec_rows],
    out_shape=[jax.ShapeDtypeStruct((_N, _D), jnp.float32)] * 2,
)

_tc_out = pl.pallas_call(
    _tc_out_body,
    grid=(_GRID,),
    in_specs=[_spec_rows, _spec_agg, _spec_den],
    out_specs=_spec_rows,
    out_shape=jax.ShapeDtypeStruct((_N, _D), jnp.float32),
)


def kernel(x, edge_index, edge_weight, W0, b0, W1, b1):
    dst = edge_index[0]
    src = edge_index[1]
    dsts = dst.reshape(_NW, _NCH, _CH)
    b0r = b0.reshape(1, _D)
    b1r = b1.reshape(1, _D)

    z0, y = _tc_in(x, W0, W1, b0r, b1r)
    msg1, den = _edge_k_denom(y, src, dsts, edge_weight)
    den = den.reshape(_NC, _DENP)
    z02, y2 = _tc_mid(z0, msg1, den, W0, W1, b0r, b1r)
    msg2, _ = _edge_k_denom(y2, src, dsts, edge_weight)
    return _tc_out(z02, msg2, den)


# speedup vs baseline: 15.7131x; 1.6704x over previous
import functools

import jax
import jax.numpy as jnp
from jax import lax
from jax.experimental import pallas as pl
from jax.experimental.pallas import tpu as pltpu
from jax.experimental.pallas import tpu_sc as plsc

_N = 10000
_E = 320000
_D = 128
_NC = 2
_NS = 16
_NW = _NC * _NS
_EPT = _E // _NW
_CH = 80
_NCH = _EPT // _CH
_ZR = 80
_RB = 640
_BN = 1024
_GRID = (_N + _BN - 1) // _BN
_DENP = _BN * _GRID


_GDN = lax.GatherDimensionNumbers(offset_dims=(), collapsed_slice_dims=(0,),
                                  start_index_map=(0,))


def _lane_bcast(v, lane):
    idx = jnp.full((16, 1), lane, jnp.int32)
    return lax.gather(v, idx, _GDN, (1,),
                      mode=lax.GatherScatterMode.PROMISE_IN_BOUNDS)


def _edge_body(with_denom, *refs):
    if with_denom:
        (y_hbm, src_hbm, dst_hbm, w_hbm, msg_hbm, den_hbm,
         src_v, dstb, w_v, rbuf, zv, gs0, gs1, ds0, ds1, msg_sp, den_sp) = refs
    else:
        (y_hbm, src_hbm, dst_hbm, w_hbm, msg_hbm,
         src_v, dstb, w_v, rbuf, zv, gs0, gs1, ds0, ds1, msg_sp) = refs
    ci = lax.axis_index("c")
    si = lax.axis_index("s")
    wid = ci * _NS + si
    gsem = (gs0, gs1)
    dsem = (ds0, ds1)

    pltpu.sync_copy(src_hbm.at[pl.ds(wid * _EPT, _EPT)], src_v)
    pltpu.sync_copy(w_hbm.at[pl.ds(wid * _EPT, _EPT)], w_v)

    def _fetch(c, slot):
        pltpu.async_copy(y_hbm.at[src_v.at[pl.ds(c * _CH, _CH)]],
                         rbuf.at[slot], gsem[slot])
        pltpu.async_copy(dst_hbm.at[pl.ds(wid * _EPT + c * _CH, _CH)],
                         dstb.at[slot], dsem[slot])

    def _wait(c, slot):
        pltpu.make_async_copy(y_hbm.at[src_v.at[pl.ds(c * _CH, _CH)]],
                              rbuf.at[slot], gsem[slot]).wait()
        pltpu.make_async_copy(dst_hbm.at[pl.ds(wid * _EPT + c * _CH, _CH)],
                              dstb.at[slot], dsem[slot]).wait()

    _fetch(0, 1)

    zero16 = jnp.zeros((16,), jnp.float32)

    def _zrow(i, _):
        for r in range(_D // 16):
            rbuf[0, i, pl.ds(r * 16, 16)] = zero16
        return ()

    lax.fori_loop(0, _ZR, _zrow, ())

    @pl.when(si < _NS - 1)
    def _():
        for b in range(_RB // _ZR):
            pltpu.sync_copy(rbuf.at[0], msg_sp.at[pl.ds(si * _RB + b * _ZR, _ZR)])

    @pl.when(si == _NS - 1)
    def _():
        base = (_NS - 1) * _RB
        for b in range(400 // _ZR):
            pltpu.sync_copy(rbuf.at[0], msg_sp.at[pl.ds(base + b * _ZR, _ZR)])

    if with_denom:
        def _zv(i, _):
            zv[pl.ds(i * 16, 16)] = zero16
            return ()
        lax.fori_loop(0, _RB // 16, _zv, ())

        @pl.when(si < _NS - 1)
        def _():
            pltpu.sync_copy(zv, den_sp.at[pl.ds(si * _RB, _RB)])

        @pl.when(si == _NS - 1)
        def _():
            pltpu.sync_copy(zv.at[pl.ds(0, 400)],
                            den_sp.at[pl.ds((_NS - 1) * _RB, 400)])
    plsc.subcore_barrier()

    def _process(c, slot):
        rb = rbuf.at[slot]

        def _scale(j, _):
            wv16 = w_v[pl.ds(c * _CH + j * 16, 16)]
            for e16 in range(16):
                e = j * 16 + e16
                wb = _lane_bcast(wv16, e16)
                for r in range(_D // 16):
                    rb[e, pl.ds(r * 16, 16)] = rb[e, pl.ds(r * 16, 16)] * wb
            return ()

        lax.fori_loop(0, _CH // 16, _scale, ())

        if with_denom:
            pltpu.sync_copy(w_v.at[pl.ds(c * _CH, _CH)],
                            den_sp.at[dstb.at[slot]], add=True)
        pltpu.sync_copy(rb, msg_sp.at[dstb.at[slot]], add=True)

    def _pair(i, _):
        a = 2 * i
        _fetch(a + 1, 0)
        _wait(a, 1)
        _process(a, 1)
        _fetch(a + 2, 1)
        _wait(a + 1, 0)
        _process(a + 1, 0)
        return ()

    lax.fori_loop(0, (_NCH - 1) // 2, _pair, ())
    _wait(_NCH - 1, 1)
    _process(_NCH - 1, 1)

    plsc.subcore_barrier()

    def _bounce_rows(lo):
        pltpu.sync_copy(msg_sp.at[pl.ds(lo, _ZR)], rbuf.at[0])
        pltpu.sync_copy(rbuf.at[0], msg_hbm.at[ci, pl.ds(lo, _ZR)])

    @pl.when(si < _NS - 1)
    def _():
        for b in range(_RB // _ZR):
            _bounce_rows(si * _RB + b * _ZR)

    @pl.when(si == _NS - 1)
    def _():
        base = (_NS - 1) * _RB
        for b in range(400 // _ZR):
            _bounce_rows(base + b * _ZR)

    if with_denom:
        @pl.when(si < _NS - 1)
        def _():
            pltpu.sync_copy(den_sp.at[pl.ds(si * _RB, _RB)], zv)
            pltpu.sync_copy(zv, den_hbm.at[pl.ds(ci * _DENP + si * _RB, _RB)])

        @pl.when(si == _NS - 1)
        def _():
            pltpu.sync_copy(den_sp.at[pl.ds((_NS - 1) * _RB, 400)],
                            zv.at[pl.ds(0, 400)])
            pltpu.sync_copy(zv.at[pl.ds(0, 400)],
                            den_hbm.at[pl.ds(ci * _DENP + (_NS - 1) * _RB, 400)])
            pltpu.sync_copy(zv.at[pl.ds(0, 240)],
                            den_hbm.at[pl.ds(ci * _DENP + _N, 240)])


def _make_edge(with_denom):
    mesh = plsc.VectorSubcoreMesh(core_axis_name="c", subcore_axis_name="s")
    out_type = [jax.ShapeDtypeStruct((_NC, _N, _D), jnp.float32)]
    if with_denom:
        out_type.append(jax.ShapeDtypeStruct((_NC * _DENP,), jnp.float32))
    scratch = [
        pltpu.VMEM((_EPT,), jnp.int32),
        pltpu.VMEM((2, _CH), jnp.int32),
        pltpu.VMEM((_EPT,), jnp.float32),
        pltpu.VMEM((2, _CH, _D), jnp.float32),
        pltpu.VMEM((_RB,), jnp.float32),
        pltpu.SemaphoreType.DMA,
        pltpu.SemaphoreType.DMA,
        pltpu.SemaphoreType.DMA,
        pltpu.SemaphoreType.DMA,
        pltpu.VMEM_SHARED((_N, _D), jnp.float32),
    ]
    if with_denom:
        scratch.append(pltpu.VMEM_SHARED((_N,), jnp.float32))
    return pl.kernel(functools.partial(_edge_body, with_denom),
                     out_type=out_type, mesh=mesh, scratch_types=scratch)


_edge_k_denom = _make_edge(True)


def _dotT(x, w):
    return lax.dot_general(x, w, (((1,), (1,)), ((), ())),
                           preferred_element_type=jnp.float32,
                           precision=lax.Precision.HIGHEST)


def _tc_in_body(x_ref, w0_ref, w1_ref, b0_ref, b1_ref, z0_ref, y_ref):
    x = x_ref[...]
    z0_ref[...] = _dotT(x, w0_ref[...]) + b0_ref[...] + b1_ref[...]
    y_ref[...] = _dotT(x, w1_ref[...])


def _rinv_block(den_ref):
    i = pl.program_id(0)
    den = den_ref[:, pl.ds(i * _BN, _BN)]
    return 1.0 / (jnp.sum(den, axis=0)[:, None] + 1e-9)


def _tc_mid_body(z0_ref, agg_ref, den_ref, w0_ref, w1_ref, b0_ref, b1_ref,
                 z02_ref, y2_ref):
    rinv = _rinv_block(den_ref)
    h = jnp.maximum(z0_ref[...] + (agg_ref[0] + agg_ref[1]) * rinv, 0.0)
    z02_ref[...] = _dotT(h, w0_ref[...]) + b0_ref[...] + b1_ref[...]
    y2_ref[...] = _dotT(h, w1_ref[...])


def _tc_out_body(z0_ref, agg_ref, den_ref, out_ref):
    rinv = _rinv_block(den_ref)
    out_ref[...] = jnp.maximum(z0_ref[...] + (agg_ref[0] + agg_ref[1]) * rinv, 0.0)


_spec_rows = pl.BlockSpec((_BN, _D), lambda i: (i, 0))
_spec_w = pl.BlockSpec((_D, _D), lambda i: (0, 0))
_spec_b = pl.BlockSpec((1, _D), lambda i: (0, 0))
_spec_agg = pl.BlockSpec((_NC, _BN, _D), lambda i: (0, i, 0))
_spec_den = pl.BlockSpec((_NC, _DENP), lambda i: (0, 0))

_tc_in = pl.pallas_call(
    _tc_in_body,
    grid=(_GRID,),
    in_specs=[_spec_rows, _spec_w, _spec_w, _spec_b, _spec_b],
    out_specs=[_spec_rows, _spec_rows],
    out_shape=[jax.ShapeDtypeStruct((_N, _D), jnp.float32)] * 2,
)

_tc_mid = pl.pallas_call(
    _tc_mid_body,
    grid=(_GRID,),
    in_specs=[_spec_rows, _spec_agg, _spec_den, _spec_w, _spec_w, _spec_b, _spec_b],
    out_specs=[_spec_rows, _spec_rows],
    out_shape=[jax.ShapeDtypeStruct((_N, _D), jnp.float32)] * 2,
)

_tc_out = pl.pallas_call(
    _tc_out_body,
    grid=(_GRID,),
    in_specs=[_spec_rows, _spec_agg, _spec_den],
    out_specs=_spec_rows,
    out_shape=jax.ShapeDtypeStruct((_N, _D), jnp.float32),
)


def kernel(x, edge_index, edge_weight, W0, b0, W1, b1):
    dst = edge_index[0]
    src = edge_index[1]
    b0r = b0.reshape(1, _D)
    b1r = b1.reshape(1, _D)

    z0, y = _tc_in(x, W0, W1, b0r, b1r)
    msg1, den = _edge_k_denom(y, src, dst, edge_weight)
    den = den.reshape(_NC, _DENP)
    z02, y2 = _tc_mid(z0, msg1, den, W0, W1, b0r, b1r)
    msg2, _ = _edge_k_denom(y2, src, dst, edge_weight)
    return _tc_out(z02, msg2, den)
